# Initial kernel scaffold; baseline (speedup 1.0000x reference)
import jax
import jax.numpy as jnp
from jax.experimental import pallas as pl


def kernel(x, edge_index, edge_values):
    raise NotImplementedError("write your pallas kernel here")


# speedup vs baseline: 2.5633x; 2.5633x over previous
import functools

import jax
import jax.numpy as jnp
from jax import lax
from jax.experimental import pallas as pl
from jax.experimental.pallas import tpu as pltpu
from jax.experimental.pallas import tpu_sc as plsc

_N_NODES = 10000
_N_EDGES = 320000
_D = 128
_NC = 2
_NS = 16
_LANES = 16
_DH = _D // _NC
_EPT = _N_EDGES // _NS
_CHUNK = 80
_NCHUNK = _EPT // _CHUNK
_NPAD = 10240
_RPT = _NPAD // _NS

_mesh = plsc.VectorSubcoreMesh(
    core_axis_name="c", subcore_axis_name="s", num_cores=_NC, num_subcores=_NS
)


@functools.partial(
    pl.kernel,
    out_type=jax.ShapeDtypeStruct((_NC, _NPAD, _DH), jnp.float32),
    mesh=_mesh,
    scratch_types=[
        pltpu.VMEM((_CHUNK,), jnp.int32),
        pltpu.VMEM((_CHUNK,), jnp.int32),
        pltpu.VMEM((_CHUNK + 16,), jnp.float32),
        pltpu.VMEM((_CHUNK, _DH), jnp.float32),
        pltpu.VMEM((_RPT, _DH), jnp.float32),
        pltpu.VMEM_SHARED((_NPAD, _DH), jnp.float32),
        pltpu.SemaphoreType.DMA,
    ],
    compiler_params=pltpu.CompilerParams(
        needs_layout_passes=False, use_tc_tiling_on_sc=False
    ),
)
def _gsum_sc(x_cat, erow, ecol, ev, out, cidx, ridx, val, rows, bounce, acc, sem):
    c = lax.axis_index("c")
    s = lax.axis_index("s")

    zeros16 = jnp.zeros((_LANES,), jnp.float32)

    def _zero_row(i, carry):
        for j in range(_DH // _LANES):
            bounce[i, pl.ds(j * _LANES, _LANES)] = zeros16
        return carry

    lax.fori_loop(0, _RPT, _zero_row, 0)
    pltpu.sync_copy(bounce, acc.at[pl.ds(s * _RPT, _RPT)])
    plsc.subcore_barrier()

    off = (c * _N_NODES).astype(jnp.int32)

    def _chunk(i, carry):
        base = s * _EPT + i * _CHUNK
        pltpu.sync_copy(erow.at[pl.ds(base, _CHUNK)], ridx)
        pltpu.sync_copy(ecol.at[pl.ds(base, _CHUNK)], cidx)
        pltpu.sync_copy(ev.at[pl.ds(base, _CHUNK)], val.at[pl.ds(16, _CHUNK)])
        for g in range(_CHUNK // _LANES):
            sl = pl.ds(g * _LANES, _LANES)
            cidx[sl] = cidx[sl] + off
        pltpu.async_copy(x_cat.at[cidx], rows, sem).wait()
        for e in range(_CHUNK):
            vb = plsc.load_gather(val, [jnp.full((_LANES,), 16 + e, jnp.int32)])
            for j in range(_DH // _LANES):
                sl = pl.ds(j * _LANES, _LANES)
                rows[e, sl] = rows[e, sl] * vb
        pltpu.sync_copy(rows, acc.at[ridx], add=True)
        return carry

    lax.fori_loop(0, _NCHUNK, _chunk, 0)
    plsc.subcore_barrier()

    pltpu.sync_copy(acc.at[pl.ds(s * _RPT, _RPT)], bounce)
    pltpu.sync_copy(bounce, out.at[c, pl.ds(s * _RPT, _RPT)])


def kernel(x, edge_index, edge_values):
    x_cat = jnp.concatenate([x[:, :_DH], x[:, _DH:]], axis=0)
    out = _gsum_sc(x_cat, edge_index[0], edge_index[1], edge_values)
    return jnp.concatenate([out[0, :_N_NODES], out[1, :_N_NODES]], axis=1)


# speedup vs baseline: 8.9204x; 3.4801x over previous
import functools

import jax
import jax.numpy as jnp
from jax import lax
from jax.experimental import pallas as pl
from jax.experimental.pallas import tpu as pltpu
from jax.experimental.pallas import tpu_sc as plsc

_N_NODES = 10000
_N_EDGES = 320000
_D = 128
_NC = 2
_NS = 16
_LANES = 16
_DH = _D // _NC
_EPT = _N_EDGES // _NS
_CHUNK = 80
_NCHUNK = _EPT // _CHUNK
_NPAD = 10240
_RPT = _NPAD // _NS
_NG = 4

_mesh = plsc.VectorSubcoreMesh(
    core_axis_name="c", subcore_axis_name="s", num_cores=_NC, num_subcores=_NS
)


@functools.partial(
    pl.kernel,
    out_type=jax.ShapeDtypeStruct((_NC, _NPAD, _DH), jnp.float32),
    mesh=_mesh,
    scratch_types=[
        pltpu.VMEM((_NCHUNK, _CHUNK), jnp.int32),
        pltpu.VMEM((_NCHUNK, _CHUNK), jnp.int32),
        pltpu.VMEM((_NCHUNK, _CHUNK), jnp.float32),
        pltpu.VMEM((_CHUNK, _DH), jnp.float32),
        pltpu.VMEM((_CHUNK, _DH), jnp.float32),
        pltpu.VMEM((_CHUNK, _DH), jnp.float32),
        pltpu.VMEM((_CHUNK, _DH), jnp.float32),
        pltpu.VMEM_SHARED((_NPAD, _DH), jnp.float32),
        pltpu.SemaphoreType.DMA,
        pltpu.SemaphoreType.DMA,
        pltpu.SemaphoreType.DMA,
        pltpu.SemaphoreType.DMA,
        pltpu.SemaphoreType.DMA,
        pltpu.SemaphoreType.DMA,
        pltpu.SemaphoreType.DMA,
        pltpu.SemaphoreType.DMA,
    ],
    compiler_params=pltpu.CompilerParams(
        needs_layout_passes=False, use_tc_tiling_on_sc=False
    ),
)
def _gsum_sc(x_cat, erow3, ecol3a, ecol3b, ev3, out, ridx, cidx, val,
             rows0, rows1, rows2, rows3,
             acc, sg0, sg1, sg2, sg3, ss0, ss1, ss2, ss3):
    c = lax.axis_index("c")
    s = lax.axis_index("s")
    rows = (rows0, rows1, rows2, rows3)
    semg = (sg0, sg1, sg2, sg3)
    sems = (ss0, ss1, ss2, ss3)

    pltpu.async_copy(erow3.at[s], ridx, sg0)

    @pl.when(c == 0)
    def _():
        pltpu.async_copy(ecol3a.at[s], cidx, sg1)

    @pl.when(c == 1)
    def _():
        pltpu.async_copy(ecol3b.at[s], cidx, sg1)

    pltpu.async_copy(ev3.at[s], val, sg2)

    zeros16 = jnp.zeros((_LANES,), jnp.float32)

    def _zero_row(i, carry):
        for j in range(_DH // _LANES):
            rows0[i, pl.ds(j * _LANES, _LANES)] = zeros16
        return carry

    lax.fori_loop(0, _CHUNK, _zero_row, 0)
    for k in range(_RPT // _CHUNK):
        pltpu.sync_copy(rows0, acc.at[pl.ds(s * _RPT + k * _CHUNK, _CHUNK)])

    pltpu.make_async_copy(erow3.at[s], ridx, sg0).wait()
    pltpu.make_async_copy(ecol3a.at[s], cidx, sg1).wait()
    pltpu.make_async_copy(ev3.at[s], val, sg2).wait()
    plsc.subcore_barrier()

    def _issue_gather(ck, g):
        pltpu.async_copy(x_cat.at[cidx.at[ck]], rows[g], semg[g])

    def _wait_gather(g):
        pltpu.make_async_copy(x_cat.at[cidx.at[0]], rows[g], semg[g]).wait()

    def _issue_scatter(ck, g):
        pltpu.async_copy(rows[g], acc.at[ridx.at[ck]], sems[g], add=True)

    def _wait_scatter(g):
        pltpu.make_async_copy(rows[g], acc.at[ridx.at[0]], sems[g]).wait()

    def _scale(ck, g):
        rbuf = rows[g]
        i0 = jnp.full((_LANES,), ck, jnp.int32)

        def _grp(eb, carry):
            for k in range(8):
                e = eb * 8 + k
                vb = plsc.load_gather(val, [i0, jnp.full((_LANES,), e, jnp.int32)])
                for j in range(_DH // _LANES):
                    sl = pl.ds(j * _LANES, _LANES)
                    rbuf[e, sl] = rbuf[e, sl] * vb
            return carry

        lax.fori_loop(0, _CHUNK // 8, _grp, 0)

    _issue_gather(0, 0)
    _issue_gather(1, 1)
    for cp in (0, 1):
        g = cp % _NG
        _issue_gather(cp + 2, (cp + 2) % _NG)
        _wait_gather(g)
        _scale(cp, g)
        _issue_scatter(cp, g)

    def _quad(t, carry):
        for j in range(4):
            ck = 2 + t * 4 + j
            g = (2 + j) % _NG
            w = (4 + j) % _NG
            _wait_scatter(w)

            @pl.when(ck + 2 < _NCHUNK)
            def _():
                _issue_gather(ck + 2, w)

            _wait_gather(g)
            _scale(ck, g)
            _issue_scatter(ck, g)
        return carry

    lax.fori_loop(0, (_NCHUNK - 2) // 4, _quad, 0)

    _wait_scatter(0)
    _wait_scatter(1)
    plsc.subcore_barrier()

    pltpu.sync_copy(acc.at[pl.ds(s * _RPT, _RPT)], out.at[c, pl.ds(s * _RPT, _RPT)])


def kernel(x, edge_index, edge_values):
    x_cat = jnp.concatenate([x[:, :_DH], x[:, _DH:]], axis=0)
    erow3 = edge_index[0].reshape(_NS, _NCHUNK, _CHUNK)
    ecol3a = edge_index[1].reshape(_NS, _NCHUNK, _CHUNK)
    ecol3b = ecol3a + _N_NODES
    ev3 = edge_values.reshape(_NS, _NCHUNK, _CHUNK)
    out = _gsum_sc(x_cat, erow3, ecol3a, ecol3b, ev3)
    return jnp.concatenate([out[0, :_N_NODES], out[1, :_N_NODES]], axis=1)
